# baseline (speedup 1.0000x reference)
import jax
import jax.numpy as jnp
from jax import lax
from jax.experimental import pallas as pl
from jax.experimental.pallas import tpu as pltpu
from jax.experimental.pallas import tpu_sc as plsc

NN = 10000
EE = 320000
DIN = 128
DEMB = 32
DTGT = 10
NCORES = 2
NSUB = 16
LANES = 16

NPAD = 10240
RPT = NPAD // NSUB
CHUNK = 128
ROWS_PER_TILE = 160
EPAD = NSUB * ROWS_PER_TILE * CHUNK
EROWS = EPAD // CHUNK

_SC_MESH = dict(core_axis_name="c", subcore_axis_name="s",
                num_cores=NCORES, num_subcores=NSUB)


def _deg_body(ei_hbm, ones_hbm, zeros_hbm, deg_hbm,
              idx_v, ones_v, zbuf_v, deg_sh):
    c = lax.axis_index("c")
    s = lax.axis_index("s")
    pltpu.sync_copy(ones_hbm, ones_v)
    pltpu.sync_copy(zeros_hbm, zbuf_v)
    pltpu.sync_copy(zbuf_v, deg_sh.at[pl.ds(s * RPT, RPT)])
    pltpu.sync_copy(ei_hbm.at[c, pl.ds(s * ROWS_PER_TILE, ROWS_PER_TILE)], idx_v)
    plsc.subcore_barrier()

    def chunk(j, carry):
        pltpu.sync_copy(ones_v, deg_sh.at[idx_v.at[j]], add=True)
        return carry

    lax.fori_loop(0, ROWS_PER_TILE, chunk, 0)
    plsc.subcore_barrier()
    pltpu.sync_copy(deg_sh.at[pl.ds(s * RPT, RPT)], zbuf_v)
    pltpu.sync_copy(zbuf_v, deg_hbm.at[c, pl.ds(s * RPT, RPT)])


_deg_kernel = pl.kernel(
    _deg_body,
    out_type=jax.ShapeDtypeStruct((NCORES, NPAD, LANES), jnp.float32),
    mesh=plsc.VectorSubcoreMesh(**_SC_MESH),
    compiler_params=pltpu.CompilerParams(use_tc_tiling_on_sc=False),
    scratch_types=[
        pltpu.VMEM((ROWS_PER_TILE, CHUNK), jnp.int32),
        pltpu.VMEM((CHUNK, LANES), jnp.float32),
        pltpu.VMEM((RPT, LANES), jnp.float32),
        pltpu.VMEM_SHARED((NPAD, LANES), jnp.float32),
    ],
)


def _proj_body(x_ref, wa_ref, wo_ref, wi_ref, brow_ref,
               a_ref, pg_ref):
    xv = x_ref[...]
    a_ref[...] = (jnp.dot(xv, wa_ref[...], preferred_element_type=jnp.float32)
                  + brow_ref[...])
    pg_ref[0, :, :] = jnp.dot(xv, wo_ref[...],
                              preferred_element_type=jnp.float32)
    pg_ref[1, :, :] = jnp.dot(xv, wi_ref[...],
                              preferred_element_type=jnp.float32)


_BLKP = 1280


def _proj_call(x, wa, wo, wi, brow):
    return pl.pallas_call(
        _proj_body,
        grid=(NPAD // _BLKP,),
        in_specs=[
            pl.BlockSpec((_BLKP, DIN), lambda i: (i, 0)),
            pl.BlockSpec((DIN, 2 * DEMB), lambda i: (0, 0)),
            pl.BlockSpec((DIN, 2 * DEMB), lambda i: (0, 0)),
            pl.BlockSpec((DIN, 2 * DEMB), lambda i: (0, 0)),
            pl.BlockSpec((1, 2 * DEMB), lambda i: (0, 0)),
        ],
        out_specs=[
            pl.BlockSpec((_BLKP, 2 * DEMB), lambda i: (i, 0)),
            pl.BlockSpec((2, _BLKP, 2 * DEMB), lambda i: (0, i, 0)),
        ],
        out_shape=[
            jax.ShapeDtypeStruct((NPAD, 2 * DEMB), jnp.float32),
            jax.ShapeDtypeStruct((2, NPAD, 2 * DEMB), jnp.float32),
        ],
    )(x, wa, wo, wi, brow)


_M = 2
_D = 1
SCHUNK = 256
SROWS = EPAD // SCHUNK // NSUB
_NSTEP = SROWS


def _scatter_body(ei_hbm, pg_hbm, deg_hbm, s_hbm, pgs_hbm,
                  idxs_v, idxd_v, rows_v, zbuf_v, degb_v, gsem, ssem, acc_sh):
    c = lax.axis_index("c")
    s = lax.axis_index("s")
    row0 = s * SROWS
    for q in range(4):
        nsl = pl.ds((c * NPAD) + s * RPT + q * (RPT // 4), RPT // 4)
        dsl = pl.ds(s * RPT + q * (RPT // 4), RPT // 4)
        pltpu.sync_copy(pg_hbm.at[nsl], zbuf_v)
        pltpu.sync_copy(deg_hbm.at[c, dsl], degb_v)

        def scale_row(r, carry):
            inv = 1.0 / jnp.maximum(degb_v[r, :], 1.0)
            for g in range(2 * DEMB // LANES):
                gsl = pl.ds(g * LANES, LANES)
                zbuf_v[r, gsl] = zbuf_v[r, gsl] * inv
            return carry

        lax.fori_loop(0, RPT // 4, scale_row, 0)
        pltpu.sync_copy(zbuf_v, pgs_hbm.at[nsl])
    def zero_row(r, carry):
        for g in range(2 * DEMB // LANES):
            gsl = pl.ds(g * LANES, LANES)
            zbuf_v[r, gsl] = jnp.zeros((LANES,), jnp.float32)
        return carry

    lax.fori_loop(0, RPT // 4, zero_row, 0)
    for q in range(4):
        pltpu.sync_copy(zbuf_v, acc_sh.at[pl.ds(s * RPT + q * (RPT // 4),
                                                RPT // 4)])
    pltpu.sync_copy(ei_hbm.at[c, pl.ds(row0, SROWS)], idxs_v)
    pltpu.sync_copy(ei_hbm.at[1 - c, pl.ds(row0, SROWS)], idxd_v)
    shift = c * NPAD

    def shift_row(r, carry):
        for g in range(SCHUNK // LANES):
            sl = pl.ds(g * LANES, LANES)
            idxs_v[r, sl] = idxs_v[r, sl] + shift
        return carry

    lax.fori_loop(0, SROWS, shift_row, 0)
    plsc.subcore_barrier()

    def start_gather(k, m):
        pltpu.async_copy(pgs_hbm.at[idxs_v.at[k]], rows_v.at[m], gsem.at[m])

    def wait_gather(m):
        pltpu.make_async_copy(pgs_hbm.at[idxs_v.at[0]], rows_v.at[m],
                              gsem.at[m]).wait()

    def start_scatter(j, m):
        pltpu.async_copy(rows_v.at[m], acc_sh.at[idxd_v.at[j]], ssem.at[m],
                         add=True)

    def wait_scatter(m):
        pltpu.make_async_copy(rows_v.at[m], acc_sh.at[idxd_v.at[0]],
                              ssem.at[m]).wait()

    for b in range(_D):
        start_gather(b, b)
    for j in range(_D):
        start_gather(j + _D, j + _D)
        wait_gather(j)
        start_scatter(j, j)

    def steady(jj, carry):
        for b in range(_M):
            j = _D + jj * _M + b
            k = j + _D
            wait_scatter(b)
            start_gather(k, b)
            m = (_D + b) % _M
            wait_gather(m)
            start_scatter(j, m)
        return carry

    lax.fori_loop(0, (_NSTEP - 2 * _D) // _M, steady, 0)

    for j in range(_NSTEP - _D, _NSTEP):
        m = j % _M
        wait_gather(m)
        start_scatter(j, m)
    for m in range(_M):
        wait_scatter(m)

    plsc.subcore_barrier()
    for q in range(4):
        sl = pl.ds(s * RPT + q * (RPT // 4), RPT // 4)
        pltpu.sync_copy(acc_sh.at[sl], zbuf_v)
        pltpu.sync_copy(zbuf_v, s_hbm.at[c, sl])


_scatter_kernel = pl.kernel(
    _scatter_body,
    out_type=[
        jax.ShapeDtypeStruct((NCORES, NPAD, 2 * DEMB), jnp.float32),
        jax.ShapeDtypeStruct((NCORES * NPAD, 2 * DEMB), jnp.float32),
    ],
    mesh=plsc.VectorSubcoreMesh(**_SC_MESH),
    compiler_params=pltpu.CompilerParams(use_tc_tiling_on_sc=False),
    scratch_types=[
        pltpu.VMEM((SROWS, SCHUNK), jnp.int32),
        pltpu.VMEM((SROWS, SCHUNK), jnp.int32),
        pltpu.VMEM((_M, SCHUNK, 2 * DEMB), jnp.float32),
        pltpu.VMEM((RPT // 4, 2 * DEMB), jnp.float32),
        pltpu.VMEM((RPT // 4, LANES), jnp.float32),
        pltpu.SemaphoreType.DMA((_M,)),
        pltpu.SemaphoreType.DMA((_M,)),
        pltpu.VMEM_SHARED((NPAD, 2 * DEMB), jnp.float32),
    ],
)


def _out_body(a_ref, s_ref, wp_ref, bp_ref, out_ref, h_ref):
    g = a_ref[...] + s_ref[0] + s_ref[1]
    z = jax.nn.sigmoid(g[:, :DEMB])
    ht = jnp.tanh(g[:, DEMB:])
    h = jnp.maximum((1.0 - z) * ht, 0.0)
    h_ref[...] = h
    out_ref[...] = (jnp.dot(h, wp_ref[...], preferred_element_type=jnp.float32)
                    + bp_ref[...])


_BLKO = 2000


def _out_call(a, svec, wp, bprow):
    return pl.pallas_call(
        _out_body,
        grid=(NN // _BLKO,),
        in_specs=[
            pl.BlockSpec((_BLKO, 2 * DEMB), lambda i: (i, 0)),
            pl.BlockSpec((2, _BLKO, 2 * DEMB), lambda i: (0, i, 0)),
            pl.BlockSpec((DEMB, DTGT), lambda i: (0, 0)),
            pl.BlockSpec((1, DTGT), lambda i: (0, 0)),
        ],
        out_specs=[
            pl.BlockSpec((_BLKO, DTGT), lambda i: (i, 0)),
            pl.BlockSpec((_BLKO, DEMB), lambda i: (i, 0)),
        ],
        out_shape=[
            jax.ShapeDtypeStruct((NN, DTGT), jnp.float32),
            jax.ShapeDtypeStruct((NN, DEMB), jnp.float32),
        ],
    )(a, svec, wp, bprow)


def kernel(x, edge_index, Wz, bz, Wr, br, Wh, bh, Wp, bp):
    del Wr, br
    f32 = jnp.float32
    wa = jnp.concatenate([Wz[0, 0, :DIN] + Wz[1, 0, :DIN],
                          Wh[0, 0, :DIN] + Wh[1, 0, :DIN]], axis=1)
    wo = jnp.concatenate([Wz[0, 1, :DIN], Wh[0, 1, :DIN]], axis=1)
    wi = jnp.concatenate([Wz[1, 1, :DIN], Wh[1, 1, :DIN]], axis=1)
    brow = jnp.concatenate([bz, bh])[None, :]

    x_pad = jnp.pad(x, ((0, NPAD - NN), (0, 0)))
    ei_pad = jnp.pad(edge_index, ((0, 0), (0, EPAD - EE)),
                     constant_values=NN)
    ei3 = ei_pad.reshape(2, EROWS, CHUNK)
    ones_c = jnp.ones((CHUNK, LANES), f32)
    zeros16 = jnp.zeros((RPT, LANES), f32)

    deg = _deg_kernel(ei3, ones_c, zeros16)
    a, pg = _proj_call(x_pad, wa, wo, wi, brow)
    ei3s = ei_pad.reshape(2, EPAD // SCHUNK, SCHUNK)
    svec, _pgs = _scatter_kernel(ei3s, pg.reshape(2 * NPAD, 2 * DEMB), deg)
    out, h = _out_call(a, svec, Wp, bp[None, :])
    return (out, h)


# speedup vs baseline: 1.0721x; 1.0721x over previous
import functools

import jax
import jax.numpy as jnp
from jax import lax
from jax.experimental import pallas as pl
from jax.experimental.pallas import tpu as pltpu
from jax.experimental.pallas import tpu_sc as plsc

NN = 10000
EE = 320000
DIN = 128
DEMB = 32
DTGT = 10
NCORES = 2
NSUB = 16
LANES = 16

NPAD = 10240
RPT = NPAD // NSUB
CHUNK = 128
ROWS_PER_TILE = 160
EPAD = NSUB * ROWS_PER_TILE * CHUNK
EROWS = EPAD // CHUNK

_SC_MESH = dict(core_axis_name="c", subcore_axis_name="s",
                num_cores=NCORES, num_subcores=NSUB)


DROWS = NPAD // LANES


def _deg_body(ei_hbm, deg_hbm, idx_v, acc_v, iota_v, deg_sh):
    c = lax.axis_index("c")
    s = lax.axis_index("s")
    pltpu.sync_copy(ei_hbm.at[c, pl.ds(s * ROWS_PER_TILE, ROWS_PER_TILE)], idx_v)

    def zrow(k, carry):
        acc_v[k, :] = jnp.zeros((LANES,), jnp.float32)
        return carry

    lax.fori_loop(0, DROWS, zrow, 0)
    zsl = pl.ds(s * (DROWS // NSUB), DROWS // NSUB)
    base = lax.iota(jnp.int32, LANES)
    for r in range(DROWS // CHUNK):
        for g in range(CHUNK // LANES):
            iota_v[r, pl.ds(g * LANES, LANES)] = base + (r * CHUNK + g * LANES)
    pltpu.sync_copy(acc_v.at[zsl], deg_sh.at[zsl])
    plsc.subcore_barrier()

    ones16 = jnp.ones((LANES,), jnp.float32)

    def hrow(r, carry):
        for g in range(CHUNK // LANES):
            n = idx_v[r, pl.ds(g * LANES, LANES)]
            plsc.addupdate_scatter(
                acc_v, [lax.shift_right_logical(n, 4),
                        lax.bitwise_and(n, 15)], ones16)
        return carry

    lax.fori_loop(0, ROWS_PER_TILE, hrow, 0)
    plsc.subcore_barrier()
    for r in range(DROWS // CHUNK):
        pltpu.sync_copy(acc_v.at[pl.ds(r * CHUNK, CHUNK)],
                        deg_sh.at[iota_v.at[r]], add=True)
    plsc.subcore_barrier()
    pltpu.sync_copy(deg_sh.at[zsl], acc_v.at[zsl])
    pltpu.sync_copy(acc_v.at[zsl], deg_hbm.at[c, zsl])


_deg_kernel = pl.kernel(
    _deg_body,
    out_type=jax.ShapeDtypeStruct((NCORES, DROWS, LANES), jnp.float32),
    mesh=plsc.VectorSubcoreMesh(**_SC_MESH),
    compiler_params=pltpu.CompilerParams(use_tc_tiling_on_sc=False,
                                         needs_layout_passes=False),
    scratch_types=[
        pltpu.VMEM((ROWS_PER_TILE, CHUNK), jnp.int32),
        pltpu.VMEM((DROWS, LANES), jnp.float32),
        pltpu.VMEM((DROWS // CHUNK, CHUNK), jnp.int32),
        pltpu.VMEM_SHARED((DROWS, LANES), jnp.float32),
    ],
)


def _proj_body(x_ref, wa_ref, wo_ref, wi_ref, brow_ref, deg_ref,
               a_ref, pg_ref):
    xv = x_ref[...]
    do = jnp.maximum(deg_ref[0], 1.0)
    di = jnp.maximum(deg_ref[1], 1.0)
    a_ref[...] = (jnp.dot(xv, wa_ref[...], preferred_element_type=jnp.float32)
                  + brow_ref[...])
    pg_ref[0, :, :] = jnp.dot(xv, wo_ref[...],
                              preferred_element_type=jnp.float32) / do
    pg_ref[1, :, :] = jnp.dot(xv, wi_ref[...],
                              preferred_element_type=jnp.float32) / di


_BLKP = 1280


def _proj_call(x, wa, wo, wi, brow, deg):
    return pl.pallas_call(
        _proj_body,
        grid=(NPAD // _BLKP,),
        in_specs=[
            pl.BlockSpec((_BLKP, DIN), lambda i: (i, 0)),
            pl.BlockSpec((DIN, 2 * DEMB), lambda i: (0, 0)),
            pl.BlockSpec((DIN, 2 * DEMB), lambda i: (0, 0)),
            pl.BlockSpec((DIN, 2 * DEMB), lambda i: (0, 0)),
            pl.BlockSpec((1, 2 * DEMB), lambda i: (0, 0)),
            pl.BlockSpec((2, _BLKP, 1), lambda i: (0, i, 0)),
        ],
        out_specs=[
            pl.BlockSpec((_BLKP, 2 * DEMB), lambda i: (i, 0)),
            pl.BlockSpec((2, _BLKP, 2 * DEMB), lambda i: (0, i, 0)),
        ],
        out_shape=[
            jax.ShapeDtypeStruct((NPAD, 2 * DEMB), jnp.float32),
            jax.ShapeDtypeStruct((2, NPAD, 2 * DEMB), jnp.float32),
        ],
    )(x, wa, wo, wi, brow, deg)


_M = 2
_D = 1
SCHUNK = 256
SROWS = EPAD // SCHUNK // NSUB
_NSTEP = SROWS


def _scatter_body(ei_hbm, pg_hbm, zeros_hbm, s_hbm,
                  idxs_v, idxd_v, rows_v, zbuf_v, gsem, ssem, acc_sh):
    c = lax.axis_index("c")
    s = lax.axis_index("s")
    row0 = s * SROWS
    pltpu.sync_copy(zeros_hbm, zbuf_v)
    for q in range(4):
        pltpu.sync_copy(zbuf_v, acc_sh.at[pl.ds(s * RPT + q * (RPT // 4),
                                                RPT // 4)])
    pltpu.sync_copy(ei_hbm.at[c, pl.ds(row0, SROWS)], idxs_v)
    pltpu.sync_copy(ei_hbm.at[1 - c, pl.ds(row0, SROWS)], idxd_v)
    shift = c * NPAD

    def shift_row(r, carry):
        for g in range(SCHUNK // LANES):
            sl = pl.ds(g * LANES, LANES)
            idxs_v[r, sl] = idxs_v[r, sl] + shift
        return carry

    lax.fori_loop(0, SROWS, shift_row, 0)
    plsc.subcore_barrier()

    def start_gather(k, m):
        pltpu.async_copy(pg_hbm.at[idxs_v.at[k]], rows_v.at[m], gsem.at[m])

    def wait_gather(m):
        pltpu.make_async_copy(pg_hbm.at[idxs_v.at[0]], rows_v.at[m],
                              gsem.at[m]).wait()

    def start_scatter(j, m):
        pltpu.async_copy(rows_v.at[m], acc_sh.at[idxd_v.at[j]], ssem.at[m],
                         add=True)

    def wait_scatter(m):
        pltpu.make_async_copy(rows_v.at[m], acc_sh.at[idxd_v.at[0]],
                              ssem.at[m]).wait()

    for b in range(_D):
        start_gather(b, b)
    for j in range(_D):
        start_gather(j + _D, j + _D)
        wait_gather(j)
        start_scatter(j, j)

    def steady(jj, carry):
        for b in range(_M):
            j = _D + jj * _M + b
            k = j + _D
            wait_scatter(b)
            start_gather(k, b)
            m = (_D + b) % _M
            wait_gather(m)
            start_scatter(j, m)
        return carry

    lax.fori_loop(0, (_NSTEP - 2 * _D) // _M, steady, 0)

    for j in range(_NSTEP - _D, _NSTEP):
        m = j % _M
        wait_gather(m)
        start_scatter(j, m)
    for m in range(_M):
        wait_scatter(m)

    plsc.subcore_barrier()
    for q in range(4):
        sl = pl.ds(s * RPT + q * (RPT // 4), RPT // 4)
        pltpu.sync_copy(acc_sh.at[sl], zbuf_v)
        pltpu.sync_copy(zbuf_v, s_hbm.at[c, sl])


_scatter_kernel = pl.kernel(
    _scatter_body,
    out_type=jax.ShapeDtypeStruct((NCORES, NPAD, 2 * DEMB), jnp.float32),
    mesh=plsc.VectorSubcoreMesh(**_SC_MESH),
    compiler_params=pltpu.CompilerParams(use_tc_tiling_on_sc=False),
    scratch_types=[
        pltpu.VMEM((SROWS, SCHUNK), jnp.int32),
        pltpu.VMEM((SROWS, SCHUNK), jnp.int32),
        pltpu.VMEM((_M, SCHUNK, 2 * DEMB), jnp.float32),
        pltpu.VMEM((RPT // 4, 2 * DEMB), jnp.float32),
        pltpu.SemaphoreType.DMA((_M,)),
        pltpu.SemaphoreType.DMA((_M,)),
        pltpu.VMEM_SHARED((NPAD, 2 * DEMB), jnp.float32),
    ],
)


def _out_body(a_ref, s_ref, wp_ref, bp_ref, out_ref, h_ref):
    g = a_ref[...] + s_ref[0] + s_ref[1]
    z = jax.nn.sigmoid(g[:, :DEMB])
    ht = jnp.tanh(g[:, DEMB:])
    h = jnp.maximum((1.0 - z) * ht, 0.0)
    h_ref[...] = h
    out_ref[...] = (jnp.dot(h, wp_ref[...], preferred_element_type=jnp.float32)
                    + bp_ref[...])


_BLKO = 2000


def _out_call(a, svec, wp, bprow):
    return pl.pallas_call(
        _out_body,
        grid=(NN // _BLKO,),
        in_specs=[
            pl.BlockSpec((_BLKO, 2 * DEMB), lambda i: (i, 0)),
            pl.BlockSpec((2, _BLKO, 2 * DEMB), lambda i: (0, i, 0)),
            pl.BlockSpec((DEMB, DTGT), lambda i: (0, 0)),
            pl.BlockSpec((1, DTGT), lambda i: (0, 0)),
        ],
        out_specs=[
            pl.BlockSpec((_BLKO, DTGT), lambda i: (i, 0)),
            pl.BlockSpec((_BLKO, DEMB), lambda i: (i, 0)),
        ],
        out_shape=[
            jax.ShapeDtypeStruct((NN, DTGT), jnp.float32),
            jax.ShapeDtypeStruct((NN, DEMB), jnp.float32),
        ],
    )(a, svec, wp, bprow)


def kernel(x, edge_index, Wz, bz, Wr, br, Wh, bh, Wp, bp):
    del Wr, br
    f32 = jnp.float32
    wa = jnp.concatenate([Wz[0, 0, :DIN] + Wz[1, 0, :DIN],
                          Wh[0, 0, :DIN] + Wh[1, 0, :DIN]], axis=1)
    wo = jnp.concatenate([Wz[0, 1, :DIN], Wh[0, 1, :DIN]], axis=1)
    wi = jnp.concatenate([Wz[1, 1, :DIN], Wh[1, 1, :DIN]], axis=1)
    brow = jnp.concatenate([bz, bh])[None, :]

    x_pad = jnp.pad(x, ((0, NPAD - NN), (0, 0)))
    ei_pad = jnp.pad(edge_index, ((0, 0), (0, EPAD - EE)),
                     constant_values=NN)
    ei3 = ei_pad.reshape(2, EROWS, CHUNK)
    zeros64 = jnp.zeros((RPT // 4, 2 * DEMB), f32)

    deg = _deg_kernel(ei3).reshape(2, NPAD, 1)
    a, pg = _proj_call(x_pad, wa, wo, wi, brow, deg)
    ei3s = ei_pad.reshape(2, EPAD // SCHUNK, SCHUNK)
    svec = _scatter_kernel(ei3s, pg.reshape(2 * NPAD, 2 * DEMB), zeros64)
    out, h = _out_call(a, svec, Wp, bp[None, :])
    return (out, h)


# speedup vs baseline: 1.5111x; 1.4095x over previous
import functools

import jax
import jax.numpy as jnp
from jax import lax
from jax.experimental import pallas as pl
from jax.experimental.pallas import tpu as pltpu
from jax.experimental.pallas import tpu_sc as plsc

NN = 10000
EE = 320000
DIN = 128
DEMB = 32
DTGT = 10
NCORES = 2
NSUB = 16
LANES = 16

NPAD = 10240
RPT = NPAD // NSUB
CHUNK = 128
ROWS_PER_TILE = 160
EPAD = NSUB * ROWS_PER_TILE * CHUNK
EROWS = EPAD // CHUNK

_SC_MESH = dict(core_axis_name="c", subcore_axis_name="s",
                num_cores=NCORES, num_subcores=NSUB)


DROWS = NPAD // LANES


def _deg_body(ei_hbm, deg_hbm, idx_v, acc_v, iota_v, deg_sh):
    c = lax.axis_index("c")
    s = lax.axis_index("s")
    pltpu.sync_copy(ei_hbm.at[c, pl.ds(s * ROWS_PER_TILE, ROWS_PER_TILE)], idx_v)

    def zrow(k, carry):
        acc_v[k, :] = jnp.zeros((LANES,), jnp.float32)
        return carry

    lax.fori_loop(0, DROWS, zrow, 0)
    zsl = pl.ds(s * (DROWS // NSUB), DROWS // NSUB)
    base = lax.iota(jnp.int32, LANES)
    for r in range(DROWS // CHUNK):
        for g in range(CHUNK // LANES):
            iota_v[r, pl.ds(g * LANES, LANES)] = base + (r * CHUNK + g * LANES)
    pltpu.sync_copy(acc_v.at[zsl], deg_sh.at[zsl])
    plsc.subcore_barrier()

    ones16 = jnp.ones((LANES,), jnp.float32)

    def hrow(r, carry):
        for g in range(CHUNK // LANES):
            n = idx_v[r, pl.ds(g * LANES, LANES)]
            plsc.addupdate_scatter(
                acc_v, [lax.shift_right_logical(n, 4),
                        lax.bitwise_and(n, 15)], ones16)
        return carry

    lax.fori_loop(0, ROWS_PER_TILE, hrow, 0)
    plsc.subcore_barrier()
    for r in range(DROWS // CHUNK):
        pltpu.sync_copy(acc_v.at[pl.ds(r * CHUNK, CHUNK)],
                        deg_sh.at[iota_v.at[r]], add=True)
    plsc.subcore_barrier()
    pltpu.sync_copy(deg_sh.at[zsl], acc_v.at[zsl])
    pltpu.sync_copy(acc_v.at[zsl], deg_hbm.at[c, zsl])


_deg_kernel = pl.kernel(
    _deg_body,
    out_type=jax.ShapeDtypeStruct((NCORES, DROWS, LANES), jnp.float32),
    mesh=plsc.VectorSubcoreMesh(**_SC_MESH),
    compiler_params=pltpu.CompilerParams(use_tc_tiling_on_sc=False,
                                         needs_layout_passes=False),
    scratch_types=[
        pltpu.VMEM((ROWS_PER_TILE, CHUNK), jnp.int32),
        pltpu.VMEM((DROWS, LANES), jnp.float32),
        pltpu.VMEM((DROWS // CHUNK, CHUNK), jnp.int32),
        pltpu.VMEM_SHARED((DROWS, LANES), jnp.float32),
    ],
)


def _proj_body(x_ref, wa_ref, wo_ref, wi_ref, brow_ref, deg_ref,
               a_ref, pg_ref):
    xv = x_ref[...]
    do = jnp.maximum(deg_ref[0], 1.0)
    di = jnp.maximum(deg_ref[1], 1.0)
    a_ref[...] = (jnp.dot(xv, wa_ref[...], preferred_element_type=jnp.float32)
                  + brow_ref[...])
    pg_ref[0, :, :] = jnp.dot(xv, wo_ref[...],
                              preferred_element_type=jnp.float32) / do
    pg_ref[1, :, :] = jnp.dot(xv, wi_ref[...],
                              preferred_element_type=jnp.float32) / di


_BLKP = 1280


def _proj_call(x, wa, wo, wi, brow, deg):
    return pl.pallas_call(
        _proj_body,
        grid=(NPAD // _BLKP,),
        in_specs=[
            pl.BlockSpec((_BLKP, DIN), lambda i: (i, 0)),
            pl.BlockSpec((DIN, 2 * DEMB), lambda i: (0, 0)),
            pl.BlockSpec((DIN, 2 * DEMB), lambda i: (0, 0)),
            pl.BlockSpec((DIN, 2 * DEMB), lambda i: (0, 0)),
            pl.BlockSpec((1, 2 * DEMB), lambda i: (0, 0)),
            pl.BlockSpec((2, _BLKP, 1), lambda i: (0, i, 0)),
        ],
        out_specs=[
            pl.BlockSpec((_BLKP, 2 * DEMB), lambda i: (i, 0)),
            pl.BlockSpec((2, _BLKP, 2 * DEMB), lambda i: (0, i, 0)),
        ],
        out_shape=[
            jax.ShapeDtypeStruct((NPAD, 2 * DEMB), jnp.float32),
            jax.ShapeDtypeStruct((2, NPAD, 2 * DEMB), jnp.float32),
        ],
    )(x, wa, wo, wi, brow, deg)


_M = 2
_D = 1
SCHUNK = 32
SROWS = EPAD // SCHUNK // NSUB
_NSTEP = SROWS


def _scatter_body(ei_hbm, pg_hbm, s_hbm,
                  idxs_v, idxd_v, rows_v, gsem, ssem, pgsrc_sh, acc_sh):
    c = lax.axis_index("c")
    s = lax.axis_index("s")
    row0 = s * SROWS
    pltpu.sync_copy(pg_hbm.at[pl.ds(c * NPAD + s * RPT, RPT)],
                    pgsrc_sh.at[pl.ds(s * RPT, RPT)])
    for m in range(_M):
        for r in range(SCHUNK):
            for g in range(2 * DEMB // LANES):
                rows_v[m, r, pl.ds(g * LANES, LANES)] = jnp.zeros(
                    (LANES,), jnp.float32)
    for q in range(RPT // SCHUNK):
        pltpu.sync_copy(rows_v.at[0],
                        acc_sh.at[pl.ds(s * RPT + q * SCHUNK, SCHUNK)])
    pltpu.sync_copy(ei_hbm.at[c, pl.ds(row0, SROWS)], idxs_v)
    pltpu.sync_copy(ei_hbm.at[1 - c, pl.ds(row0, SROWS)], idxd_v)
    plsc.subcore_barrier()

    def start_gather(k, m):
        pltpu.async_copy(pgsrc_sh.at[idxs_v.at[k]], rows_v.at[m], gsem.at[m])

    def wait_gather(m):
        pltpu.make_async_copy(pgsrc_sh.at[idxs_v.at[0]], rows_v.at[m],
                              gsem.at[m]).wait()

    def start_scatter(j, m):
        pltpu.async_copy(rows_v.at[m], acc_sh.at[idxd_v.at[j]], ssem.at[m],
                         add=True)

    def wait_scatter(m):
        pltpu.make_async_copy(rows_v.at[m], acc_sh.at[idxd_v.at[0]],
                              ssem.at[m]).wait()

    for b in range(_D):
        start_gather(b, b)
    for j in range(_D):
        start_gather(j + _D, j + _D)
        wait_gather(j)
        start_scatter(j, j)

    def steady(jj, carry):
        for b in range(_M):
            j = _D + jj * _M + b
            k = j + _D
            wait_scatter(b)
            start_gather(k, b)
            m = (_D + b) % _M
            wait_gather(m)
            start_scatter(j, m)
        return carry

    lax.fori_loop(0, (_NSTEP - 2 * _D) // _M, steady, 0)

    for j in range(_NSTEP - _D, _NSTEP):
        m = j % _M
        wait_gather(m)
        start_scatter(j, m)
    for m in range(_M):
        wait_scatter(m)

    plsc.subcore_barrier()
    for q in range(RPT // SCHUNK):
        sl = pl.ds(s * RPT + q * SCHUNK, SCHUNK)
        pltpu.sync_copy(acc_sh.at[sl], rows_v.at[0])
        pltpu.sync_copy(rows_v.at[0], s_hbm.at[c, sl])


_scatter_kernel = pl.kernel(
    _scatter_body,
    out_type=jax.ShapeDtypeStruct((NCORES, NPAD, 2 * DEMB), jnp.float32),
    mesh=plsc.VectorSubcoreMesh(**_SC_MESH),
    compiler_params=pltpu.CompilerParams(use_tc_tiling_on_sc=False,
                                         needs_layout_passes=False),
    scratch_types=[
        pltpu.VMEM((SROWS, SCHUNK), jnp.int32),
        pltpu.VMEM((SROWS, SCHUNK), jnp.int32),
        pltpu.VMEM((_M, SCHUNK, 2 * DEMB), jnp.float32),
        pltpu.SemaphoreType.DMA((_M,)),
        pltpu.SemaphoreType.DMA((_M,)),
        pltpu.VMEM_SHARED((NPAD, 2 * DEMB), jnp.float32),
        pltpu.VMEM_SHARED((NPAD, 2 * DEMB), jnp.float32),
    ],
)


def _out_body(a_ref, s_ref, wp_ref, bp_ref, out_ref, h_ref):
    g = a_ref[...] + s_ref[0] + s_ref[1]
    z = jax.nn.sigmoid(g[:, :DEMB])
    ht = jnp.tanh(g[:, DEMB:])
    h = jnp.maximum((1.0 - z) * ht, 0.0)
    h_ref[...] = h
    out_ref[...] = (jnp.dot(h, wp_ref[...], preferred_element_type=jnp.float32)
                    + bp_ref[...])


_BLKO = 2000


def _out_call(a, svec, wp, bprow):
    return pl.pallas_call(
        _out_body,
        grid=(NN // _BLKO,),
        in_specs=[
            pl.BlockSpec((_BLKO, 2 * DEMB), lambda i: (i, 0)),
            pl.BlockSpec((2, _BLKO, 2 * DEMB), lambda i: (0, i, 0)),
            pl.BlockSpec((DEMB, DTGT), lambda i: (0, 0)),
            pl.BlockSpec((1, DTGT), lambda i: (0, 0)),
        ],
        out_specs=[
            pl.BlockSpec((_BLKO, DTGT), lambda i: (i, 0)),
            pl.BlockSpec((_BLKO, DEMB), lambda i: (i, 0)),
        ],
        out_shape=[
            jax.ShapeDtypeStruct((NN, DTGT), jnp.float32),
            jax.ShapeDtypeStruct((NN, DEMB), jnp.float32),
        ],
    )(a, svec, wp, bprow)


def kernel(x, edge_index, Wz, bz, Wr, br, Wh, bh, Wp, bp):
    del Wr, br
    f32 = jnp.float32
    wa = jnp.concatenate([Wz[0, 0, :DIN] + Wz[1, 0, :DIN],
                          Wh[0, 0, :DIN] + Wh[1, 0, :DIN]], axis=1)
    wo = jnp.concatenate([Wz[0, 1, :DIN], Wh[0, 1, :DIN]], axis=1)
    wi = jnp.concatenate([Wz[1, 1, :DIN], Wh[1, 1, :DIN]], axis=1)
    brow = jnp.concatenate([bz, bh])[None, :]

    x_pad = jnp.pad(x, ((0, NPAD - NN), (0, 0)))
    ei_pad = jnp.pad(edge_index, ((0, 0), (0, EPAD - EE)),
                     constant_values=NN)
    ei3 = ei_pad.reshape(2, EROWS, CHUNK)
    deg = _deg_kernel(ei3).reshape(2, NPAD, 1)
    a, pg = _proj_call(x_pad, wa, wo, wi, brow, deg)
    ei3s = ei_pad.reshape(2, EPAD // SCHUNK, SCHUNK)
    svec = _scatter_kernel(ei3s, pg.reshape(2 * NPAD, 2 * DEMB))
    out, h = _out_call(a, svec, Wp, bp[None, :])
    return (out, h)


# speedup vs baseline: 1.5790x; 1.0449x over previous
import functools

import jax
import jax.numpy as jnp
from jax import lax
from jax.experimental import pallas as pl
from jax.experimental.pallas import tpu as pltpu
from jax.experimental.pallas import tpu_sc as plsc

NN = 10000
EE = 320000
DIN = 128
DEMB = 32
DTGT = 10
NCORES = 2
NSUB = 16
LANES = 16

NPAD = 10240
RPT = NPAD // NSUB
CHUNK = 128
ROWS_PER_TILE = 160
EPAD = NSUB * ROWS_PER_TILE * CHUNK
EROWS = EPAD // CHUNK

_SC_MESH = dict(core_axis_name="c", subcore_axis_name="s",
                num_cores=NCORES, num_subcores=NSUB)


DROWS = NPAD // LANES


def _deg_body(ei_hbm, deg_hbm, idx_v, acc_v, iota_v, deg_sh):
    c = lax.axis_index("c")
    s = lax.axis_index("s")
    pltpu.sync_copy(ei_hbm.at[c, pl.ds(s * ROWS_PER_TILE, ROWS_PER_TILE)], idx_v)

    def zrow(k, carry):
        acc_v[k, :] = jnp.zeros((LANES,), jnp.float32)
        return carry

    lax.fori_loop(0, DROWS, zrow, 0)
    zsl = pl.ds(s * (DROWS // NSUB), DROWS // NSUB)
    base = lax.iota(jnp.int32, LANES)
    for r in range(DROWS // CHUNK):
        for g in range(CHUNK // LANES):
            iota_v[r, pl.ds(g * LANES, LANES)] = base + (r * CHUNK + g * LANES)
    pltpu.sync_copy(acc_v.at[zsl], deg_sh.at[zsl])
    plsc.subcore_barrier()

    ones16 = jnp.ones((LANES,), jnp.float32)

    def hrow(r, carry):
        for g in range(CHUNK // LANES):
            n = idx_v[r, pl.ds(g * LANES, LANES)]
            plsc.addupdate_scatter(
                acc_v, [lax.shift_right_logical(n, 4),
                        lax.bitwise_and(n, 15)], ones16)
        return carry

    lax.fori_loop(0, ROWS_PER_TILE, hrow, 0)
    plsc.subcore_barrier()
    for r in range(DROWS // CHUNK):
        pltpu.sync_copy(acc_v.at[pl.ds(r * CHUNK, CHUNK)],
                        deg_sh.at[iota_v.at[r]], add=True)
    plsc.subcore_barrier()
    pltpu.sync_copy(deg_sh.at[zsl], acc_v.at[zsl])
    pltpu.sync_copy(acc_v.at[zsl], deg_hbm.at[c, zsl])


_deg_kernel = pl.kernel(
    _deg_body,
    out_type=jax.ShapeDtypeStruct((NCORES, DROWS, LANES), jnp.float32),
    mesh=plsc.VectorSubcoreMesh(**_SC_MESH),
    compiler_params=pltpu.CompilerParams(use_tc_tiling_on_sc=False,
                                         needs_layout_passes=False),
    scratch_types=[
        pltpu.VMEM((ROWS_PER_TILE, CHUNK), jnp.int32),
        pltpu.VMEM((DROWS, LANES), jnp.float32),
        pltpu.VMEM((DROWS // CHUNK, CHUNK), jnp.int32),
        pltpu.VMEM_SHARED((DROWS, LANES), jnp.float32),
    ],
)


def _proj_body(x_ref, wa_ref, wo_ref, wi_ref, brow_ref, deg_ref,
               a_ref, pg_ref):
    xv = x_ref[...]
    do = jnp.maximum(deg_ref[0], 1.0)
    di = jnp.maximum(deg_ref[1], 1.0)
    a_ref[...] = (jnp.dot(xv, wa_ref[...], preferred_element_type=jnp.float32)
                  + brow_ref[...])
    pg_ref[0, :, :] = jnp.dot(xv, wo_ref[...],
                              preferred_element_type=jnp.float32) / do
    pg_ref[1, :, :] = jnp.dot(xv, wi_ref[...],
                              preferred_element_type=jnp.float32) / di


_BLKP = 1280


def _proj_call(x, wa, wo, wi, brow, deg):
    return pl.pallas_call(
        _proj_body,
        grid=(NPAD // _BLKP,),
        in_specs=[
            pl.BlockSpec((_BLKP, DIN), lambda i: (i, 0)),
            pl.BlockSpec((DIN, 2 * DEMB), lambda i: (0, 0)),
            pl.BlockSpec((DIN, 2 * DEMB), lambda i: (0, 0)),
            pl.BlockSpec((DIN, 2 * DEMB), lambda i: (0, 0)),
            pl.BlockSpec((1, 2 * DEMB), lambda i: (0, 0)),
            pl.BlockSpec((2, _BLKP, 1), lambda i: (0, i, 0)),
        ],
        out_specs=[
            pl.BlockSpec((_BLKP, 2 * DEMB), lambda i: (i, 0)),
            pl.BlockSpec((2, _BLKP, 2 * DEMB), lambda i: (0, i, 0)),
        ],
        out_shape=[
            jax.ShapeDtypeStruct((NPAD, 2 * DEMB), jnp.float32),
            jax.ShapeDtypeStruct((2, NPAD, 2 * DEMB), jnp.float32),
        ],
    )(x, wa, wo, wi, brow, deg)


_M = 2
_D = 1
SCHUNK = 64
SROWS = EPAD // SCHUNK // NSUB
_NSTEP = SROWS


def _scatter_body(ei_hbm, pg_hbm, s_hbm,
                  idxs_v, idxd_v, rows_v, gsem, ssem, pgsrc_sh, acc_sh):
    c = lax.axis_index("c")
    s = lax.axis_index("s")
    row0 = s * SROWS
    pltpu.sync_copy(pg_hbm.at[pl.ds(c * NPAD + s * RPT, RPT)],
                    pgsrc_sh.at[pl.ds(s * RPT, RPT)])
    for m in range(_M):
        for r in range(SCHUNK):
            for g in range(2 * DEMB // LANES):
                rows_v[m, r, pl.ds(g * LANES, LANES)] = jnp.zeros(
                    (LANES,), jnp.float32)
    for q in range(RPT // SCHUNK):
        pltpu.sync_copy(rows_v.at[0],
                        acc_sh.at[pl.ds(s * RPT + q * SCHUNK, SCHUNK)])
    pltpu.sync_copy(ei_hbm.at[c, pl.ds(row0, SROWS)], idxs_v)
    pltpu.sync_copy(ei_hbm.at[1 - c, pl.ds(row0, SROWS)], idxd_v)
    plsc.subcore_barrier()

    def start_gather(k, m):
        pltpu.async_copy(pgsrc_sh.at[idxs_v.at[k]], rows_v.at[m], gsem.at[m])

    def wait_gather(m):
        pltpu.make_async_copy(pgsrc_sh.at[idxs_v.at[0]], rows_v.at[m],
                              gsem.at[m]).wait()

    def start_scatter(j, m):
        pltpu.async_copy(rows_v.at[m], acc_sh.at[idxd_v.at[j]], ssem.at[m],
                         add=True)

    def wait_scatter(m):
        pltpu.make_async_copy(rows_v.at[m], acc_sh.at[idxd_v.at[0]],
                              ssem.at[m]).wait()

    for b in range(_D):
        start_gather(b, b)
    for j in range(_D):
        start_gather(j + _D, j + _D)
        wait_gather(j)
        start_scatter(j, j)

    def steady(jj, carry):
        for b in range(_M):
            j = _D + jj * _M + b
            k = j + _D
            wait_scatter(b)
            start_gather(k, b)
            m = (_D + b) % _M
            wait_gather(m)
            start_scatter(j, m)
        return carry

    lax.fori_loop(0, (_NSTEP - 2 * _D) // _M, steady, 0)

    for j in range(_NSTEP - _D, _NSTEP):
        m = j % _M
        wait_gather(m)
        start_scatter(j, m)
    for m in range(_M):
        wait_scatter(m)

    plsc.subcore_barrier()
    for q in range(RPT // SCHUNK):
        sl = pl.ds(s * RPT + q * SCHUNK, SCHUNK)
        pltpu.sync_copy(acc_sh.at[sl], rows_v.at[0])
        pltpu.sync_copy(rows_v.at[0], s_hbm.at[c, sl])


_scatter_kernel = pl.kernel(
    _scatter_body,
    out_type=jax.ShapeDtypeStruct((NCORES, NPAD, 2 * DEMB), jnp.float32),
    mesh=plsc.VectorSubcoreMesh(**_SC_MESH),
    compiler_params=pltpu.CompilerParams(use_tc_tiling_on_sc=False,
                                         needs_layout_passes=False),
    scratch_types=[
        pltpu.VMEM((SROWS, SCHUNK), jnp.int32),
        pltpu.VMEM((SROWS, SCHUNK), jnp.int32),
        pltpu.VMEM((_M, SCHUNK, 2 * DEMB), jnp.float32),
        pltpu.SemaphoreType.DMA((_M,)),
        pltpu.SemaphoreType.DMA((_M,)),
        pltpu.VMEM_SHARED((NPAD, 2 * DEMB), jnp.float32),
        pltpu.VMEM_SHARED((NPAD, 2 * DEMB), jnp.float32),
    ],
)


def _out_body(a_ref, s_ref, wp_ref, bp_ref, out_ref, h_ref):
    g = a_ref[...] + s_ref[0] + s_ref[1]
    z = jax.nn.sigmoid(g[:, :DEMB])
    ht = jnp.tanh(g[:, DEMB:])
    h = jnp.maximum((1.0 - z) * ht, 0.0)
    h_ref[...] = h
    out_ref[...] = (jnp.dot(h, wp_ref[...], preferred_element_type=jnp.float32)
                    + bp_ref[...])


_BLKO = 2000


def _out_call(a, svec, wp, bprow):
    return pl.pallas_call(
        _out_body,
        grid=(NN // _BLKO,),
        in_specs=[
            pl.BlockSpec((_BLKO, 2 * DEMB), lambda i: (i, 0)),
            pl.BlockSpec((2, _BLKO, 2 * DEMB), lambda i: (0, i, 0)),
            pl.BlockSpec((DEMB, DTGT), lambda i: (0, 0)),
            pl.BlockSpec((1, DTGT), lambda i: (0, 0)),
        ],
        out_specs=[
            pl.BlockSpec((_BLKO, DTGT), lambda i: (i, 0)),
            pl.BlockSpec((_BLKO, DEMB), lambda i: (i, 0)),
        ],
        out_shape=[
            jax.ShapeDtypeStruct((NN, DTGT), jnp.float32),
            jax.ShapeDtypeStruct((NN, DEMB), jnp.float32),
        ],
    )(a, svec, wp, bprow)


def kernel(x, edge_index, Wz, bz, Wr, br, Wh, bh, Wp, bp):
    del Wr, br
    f32 = jnp.float32
    wa = jnp.concatenate([Wz[0, 0, :DIN] + Wz[1, 0, :DIN],
                          Wh[0, 0, :DIN] + Wh[1, 0, :DIN]], axis=1)
    wo = jnp.concatenate([Wz[0, 1, :DIN], Wh[0, 1, :DIN]], axis=1)
    wi = jnp.concatenate([Wz[1, 1, :DIN], Wh[1, 1, :DIN]], axis=1)
    brow = jnp.concatenate([bz, bh])[None, :]

    x_pad = jnp.pad(x, ((0, NPAD - NN), (0, 0)))
    ei_pad = jnp.pad(edge_index, ((0, 0), (0, EPAD - EE)),
                     constant_values=NN)
    ei3 = ei_pad.reshape(2, EROWS, CHUNK)
    deg = _deg_kernel(ei3).reshape(2, NPAD, 1)
    a, pg = _proj_call(x_pad, wa, wo, wi, brow, deg)
    ei3s = ei_pad.reshape(2, EPAD // SCHUNK, SCHUNK)
    svec = _scatter_kernel(ei3s, pg.reshape(2 * NPAD, 2 * DEMB))
    out, h = _out_call(a, svec, Wp, bp[None, :])
    return (out, h)


# speedup vs baseline: 1.5799x; 1.0005x over previous
import functools

import jax
import jax.numpy as jnp
from jax import lax
from jax.experimental import pallas as pl
from jax.experimental.pallas import tpu as pltpu
from jax.experimental.pallas import tpu_sc as plsc

NN = 10000
EE = 320000
DIN = 128
DEMB = 32
DTGT = 10
NCORES = 2
NSUB = 16
LANES = 16

NPAD = 10240
RPT = NPAD // NSUB
CHUNK = 128
ROWS_PER_TILE = 160
EPAD = NSUB * ROWS_PER_TILE * CHUNK
EROWS = EPAD // CHUNK

_SC_MESH = dict(core_axis_name="c", subcore_axis_name="s",
                num_cores=NCORES, num_subcores=NSUB)


DROWS = NPAD // LANES


def _deg_body(ei_hbm, deg_hbm, idx_v, acc_v, iota_v, deg_sh):
    c = lax.axis_index("c")
    s = lax.axis_index("s")
    pltpu.sync_copy(ei_hbm.at[c, pl.ds(s * ROWS_PER_TILE, ROWS_PER_TILE)], idx_v)

    def zrow(k, carry):
        acc_v[k, :] = jnp.zeros((LANES,), jnp.float32)
        return carry

    lax.fori_loop(0, DROWS, zrow, 0)
    zsl = pl.ds(s * (DROWS // NSUB), DROWS // NSUB)
    base = lax.iota(jnp.int32, LANES)
    for r in range(DROWS // CHUNK):
        for g in range(CHUNK // LANES):
            iota_v[r, pl.ds(g * LANES, LANES)] = base + (r * CHUNK + g * LANES)
    pltpu.sync_copy(acc_v.at[zsl], deg_sh.at[zsl])
    plsc.subcore_barrier()

    ones16 = jnp.ones((LANES,), jnp.float32)

    def hrow(r, carry):
        for g in range(CHUNK // LANES):
            n = idx_v[r, pl.ds(g * LANES, LANES)]
            plsc.addupdate_scatter(
                acc_v, [lax.shift_right_logical(n, 4),
                        lax.bitwise_and(n, 15)], ones16)
        return carry

    lax.fori_loop(0, ROWS_PER_TILE, hrow, 0)
    plsc.subcore_barrier()
    for r in range(DROWS // CHUNK):
        pltpu.sync_copy(acc_v.at[pl.ds(r * CHUNK, CHUNK)],
                        deg_sh.at[iota_v.at[r]], add=True)
    plsc.subcore_barrier()
    pltpu.sync_copy(deg_sh.at[zsl], acc_v.at[zsl])
    pltpu.sync_copy(acc_v.at[zsl], deg_hbm.at[c, zsl])


_deg_kernel = pl.kernel(
    _deg_body,
    out_type=jax.ShapeDtypeStruct((NCORES, DROWS, LANES), jnp.float32),
    mesh=plsc.VectorSubcoreMesh(**_SC_MESH),
    compiler_params=pltpu.CompilerParams(use_tc_tiling_on_sc=False,
                                         needs_layout_passes=False,
                                         skip_device_barrier=True),
    scratch_types=[
        pltpu.VMEM((ROWS_PER_TILE, CHUNK), jnp.int32),
        pltpu.VMEM((DROWS, LANES), jnp.float32),
        pltpu.VMEM((DROWS // CHUNK, CHUNK), jnp.int32),
        pltpu.VMEM_SHARED((DROWS, LANES), jnp.float32),
    ],
)


def _proj_body(x_ref, wa_ref, wo_ref, wi_ref, brow_ref, deg_ref,
               a_ref, pg_ref):
    xv = x_ref[...]
    do = jnp.maximum(deg_ref[0], 1.0)
    di = jnp.maximum(deg_ref[1], 1.0)
    a_ref[...] = (jnp.dot(xv, wa_ref[...], preferred_element_type=jnp.float32)
                  + brow_ref[...])
    pg_ref[0, :, :] = jnp.dot(xv, wo_ref[...],
                              preferred_element_type=jnp.float32) / do
    pg_ref[1, :, :] = jnp.dot(xv, wi_ref[...],
                              preferred_element_type=jnp.float32) / di


_BLKP = 1280


def _proj_call(x, wa, wo, wi, brow, deg):
    return pl.pallas_call(
        _proj_body,
        grid=(NPAD // _BLKP,),
        in_specs=[
            pl.BlockSpec((_BLKP, DIN), lambda i: (i, 0)),
            pl.BlockSpec((DIN, 2 * DEMB), lambda i: (0, 0)),
            pl.BlockSpec((DIN, 2 * DEMB), lambda i: (0, 0)),
            pl.BlockSpec((DIN, 2 * DEMB), lambda i: (0, 0)),
            pl.BlockSpec((1, 2 * DEMB), lambda i: (0, 0)),
            pl.BlockSpec((2, _BLKP, 1), lambda i: (0, i, 0)),
        ],
        out_specs=[
            pl.BlockSpec((_BLKP, 2 * DEMB), lambda i: (i, 0)),
            pl.BlockSpec((2, _BLKP, 2 * DEMB), lambda i: (0, i, 0)),
        ],
        out_shape=[
            jax.ShapeDtypeStruct((NPAD, 2 * DEMB), jnp.float32),
            jax.ShapeDtypeStruct((2, NPAD, 2 * DEMB), jnp.float32),
        ],
    )(x, wa, wo, wi, brow, deg)


_M = 2
_D = 1
SCHUNK = 64
SROWS = EPAD // SCHUNK // NSUB
_NSTEP = SROWS


def _scatter_body(ei_hbm, pg_hbm, s_hbm,
                  idxs_v, idxd_v, rows_v, gsem, ssem, pgsrc_sh, acc_sh):
    c = lax.axis_index("c")
    s = lax.axis_index("s")
    row0 = s * SROWS
    pltpu.sync_copy(pg_hbm.at[pl.ds(c * NPAD + s * RPT, RPT)],
                    pgsrc_sh.at[pl.ds(s * RPT, RPT)])
    for m in range(_M):
        for r in range(SCHUNK):
            for g in range(2 * DEMB // LANES):
                rows_v[m, r, pl.ds(g * LANES, LANES)] = jnp.zeros(
                    (LANES,), jnp.float32)
    for q in range(RPT // SCHUNK):
        pltpu.sync_copy(rows_v.at[0],
                        acc_sh.at[pl.ds(s * RPT + q * SCHUNK, SCHUNK)])
    pltpu.sync_copy(ei_hbm.at[c, pl.ds(row0, SROWS)], idxs_v)
    pltpu.sync_copy(ei_hbm.at[1 - c, pl.ds(row0, SROWS)], idxd_v)
    plsc.subcore_barrier()

    def start_gather(k, m):
        pltpu.async_copy(pgsrc_sh.at[idxs_v.at[k]], rows_v.at[m], gsem.at[m])

    def wait_gather(m):
        pltpu.make_async_copy(pgsrc_sh.at[idxs_v.at[0]], rows_v.at[m],
                              gsem.at[m]).wait()

    def start_scatter(j, m):
        pltpu.async_copy(rows_v.at[m], acc_sh.at[idxd_v.at[j]], ssem.at[m],
                         add=True)

    def wait_scatter(m):
        pltpu.make_async_copy(rows_v.at[m], acc_sh.at[idxd_v.at[0]],
                              ssem.at[m]).wait()

    for b in range(_D):
        start_gather(b, b)
    for j in range(_D):
        start_gather(j + _D, j + _D)
        wait_gather(j)
        start_scatter(j, j)

    def steady(jj, carry):
        for b in range(_M):
            j = _D + jj * _M + b
            k = j + _D
            wait_scatter(b)
            start_gather(k, b)
            m = (_D + b) % _M
            wait_gather(m)
            start_scatter(j, m)
        return carry

    lax.fori_loop(0, (_NSTEP - 2 * _D) // _M, steady, 0)

    for j in range(_NSTEP - _D, _NSTEP):
        m = j % _M
        wait_gather(m)
        start_scatter(j, m)
    for m in range(_M):
        wait_scatter(m)

    plsc.subcore_barrier()
    for q in range(RPT // SCHUNK):
        sl = pl.ds(s * RPT + q * SCHUNK, SCHUNK)
        pltpu.sync_copy(acc_sh.at[sl], rows_v.at[0])
        pltpu.sync_copy(rows_v.at[0], s_hbm.at[c, sl])


_scatter_kernel = pl.kernel(
    _scatter_body,
    out_type=jax.ShapeDtypeStruct((NCORES, NPAD, 2 * DEMB), jnp.float32),
    mesh=plsc.VectorSubcoreMesh(**_SC_MESH),
    compiler_params=pltpu.CompilerParams(use_tc_tiling_on_sc=False,
                                         needs_layout_passes=False,
                                         skip_device_barrier=True),
    scratch_types=[
        pltpu.VMEM((SROWS, SCHUNK), jnp.int32),
        pltpu.VMEM((SROWS, SCHUNK), jnp.int32),
        pltpu.VMEM((_M, SCHUNK, 2 * DEMB), jnp.float32),
        pltpu.SemaphoreType.DMA((_M,)),
        pltpu.SemaphoreType.DMA((_M,)),
        pltpu.VMEM_SHARED((NPAD, 2 * DEMB), jnp.float32),
        pltpu.VMEM_SHARED((NPAD, 2 * DEMB), jnp.float32),
    ],
)


def _out_body(a_ref, s_ref, wp_ref, bp_ref, out_ref, h_ref):
    g = a_ref[...] + s_ref[0] + s_ref[1]
    z = jax.nn.sigmoid(g[:, :DEMB])
    ht = jnp.tanh(g[:, DEMB:])
    h = jnp.maximum((1.0 - z) * ht, 0.0)
    h_ref[...] = h
    out_ref[...] = (jnp.dot(h, wp_ref[...], preferred_element_type=jnp.float32)
                    + bp_ref[...])


_BLKO = 2000


def _out_call(a, svec, wp, bprow):
    return pl.pallas_call(
        _out_body,
        grid=(NN // _BLKO,),
        in_specs=[
            pl.BlockSpec((_BLKO, 2 * DEMB), lambda i: (i, 0)),
            pl.BlockSpec((2, _BLKO, 2 * DEMB), lambda i: (0, i, 0)),
            pl.BlockSpec((DEMB, DTGT), lambda i: (0, 0)),
            pl.BlockSpec((1, DTGT), lambda i: (0, 0)),
        ],
        out_specs=[
            pl.BlockSpec((_BLKO, DTGT), lambda i: (i, 0)),
            pl.BlockSpec((_BLKO, DEMB), lambda i: (i, 0)),
        ],
        out_shape=[
            jax.ShapeDtypeStruct((NN, DTGT), jnp.float32),
            jax.ShapeDtypeStruct((NN, DEMB), jnp.float32),
        ],
    )(a, svec, wp, bprow)


def kernel(x, edge_index, Wz, bz, Wr, br, Wh, bh, Wp, bp):
    del Wr, br
    f32 = jnp.float32
    wa = jnp.concatenate([Wz[0, 0, :DIN] + Wz[1, 0, :DIN],
                          Wh[0, 0, :DIN] + Wh[1, 0, :DIN]], axis=1)
    wo = jnp.concatenate([Wz[0, 1, :DIN], Wh[0, 1, :DIN]], axis=1)
    wi = jnp.concatenate([Wz[1, 1, :DIN], Wh[1, 1, :DIN]], axis=1)
    brow = jnp.concatenate([bz, bh])[None, :]

    x_pad = jnp.pad(x, ((0, NPAD - NN), (0, 0)))
    ei_pad = jnp.pad(edge_index, ((0, 0), (0, EPAD - EE)),
                     constant_values=NN)
    ei3 = ei_pad.reshape(2, EROWS, CHUNK)
    deg = _deg_kernel(ei3).reshape(2, NPAD, 1)
    a, pg = _proj_call(x_pad, wa, wo, wi, brow, deg)
    ei3s = ei_pad.reshape(2, EPAD // SCHUNK, SCHUNK)
    svec = _scatter_kernel(ei3s, pg.reshape(2 * NPAD, 2 * DEMB))
    out, h = _out_call(a, svec, Wp, bp[None, :])
    return (out, h)


# speedup vs baseline: 1.5804x; 1.0003x over previous
import functools

import jax
import jax.numpy as jnp
from jax import lax
from jax.experimental import pallas as pl
from jax.experimental.pallas import tpu as pltpu
from jax.experimental.pallas import tpu_sc as plsc

NN = 10000
EE = 320000
DIN = 128
DEMB = 32
DTGT = 10
NCORES = 2
NSUB = 16
LANES = 16

NPAD = 10240
RPT = NPAD // NSUB
CHUNK = 128
ROWS_PER_TILE = 160
EPAD = NSUB * ROWS_PER_TILE * CHUNK
EROWS = EPAD // CHUNK

_SC_MESH = dict(core_axis_name="c", subcore_axis_name="s",
                num_cores=NCORES, num_subcores=NSUB)


DROWS = NPAD // LANES


def _deg_body(ei_hbm, deg_hbm, idx_v, acc_v, iota_v, deg_sh):
    c = lax.axis_index("c")
    s = lax.axis_index("s")
    pltpu.sync_copy(ei_hbm.at[c, pl.ds(s * ROWS_PER_TILE, ROWS_PER_TILE)], idx_v)

    def zrow(k, carry):
        acc_v[k, :] = jnp.zeros((LANES,), jnp.float32)
        return carry

    lax.fori_loop(0, DROWS, zrow, 0)
    zsl = pl.ds(s * (DROWS // NSUB), DROWS // NSUB)
    base = lax.iota(jnp.int32, LANES)
    for r in range(DROWS // CHUNK):
        for g in range(CHUNK // LANES):
            iota_v[r, pl.ds(g * LANES, LANES)] = base + (r * CHUNK + g * LANES)
    pltpu.sync_copy(acc_v.at[zsl], deg_sh.at[zsl])
    plsc.subcore_barrier()

    ones16 = jnp.ones((LANES,), jnp.float32)

    def hrow(r, carry):
        for g in range(CHUNK // LANES):
            n = idx_v[r, pl.ds(g * LANES, LANES)]
            plsc.addupdate_scatter(
                acc_v, [lax.shift_right_logical(n, 4),
                        lax.bitwise_and(n, 15)], ones16)
        return carry

    lax.fori_loop(0, ROWS_PER_TILE, hrow, 0)
    plsc.subcore_barrier()
    for r in range(DROWS // CHUNK):
        pltpu.sync_copy(acc_v.at[pl.ds(r * CHUNK, CHUNK)],
                        deg_sh.at[iota_v.at[r]], add=True)
    plsc.subcore_barrier()
    pltpu.sync_copy(deg_sh.at[zsl], acc_v.at[zsl])
    pltpu.sync_copy(acc_v.at[zsl], deg_hbm.at[c, zsl])


_deg_kernel = pl.kernel(
    _deg_body,
    out_type=jax.ShapeDtypeStruct((NCORES, DROWS, LANES), jnp.float32),
    mesh=plsc.VectorSubcoreMesh(**_SC_MESH),
    compiler_params=pltpu.CompilerParams(use_tc_tiling_on_sc=False,
                                         needs_layout_passes=False),
    scratch_types=[
        pltpu.VMEM((ROWS_PER_TILE, CHUNK), jnp.int32),
        pltpu.VMEM((DROWS, LANES), jnp.float32),
        pltpu.VMEM((DROWS // CHUNK, CHUNK), jnp.int32),
        pltpu.VMEM_SHARED((DROWS, LANES), jnp.float32),
    ],
)


def _proj_body(x_ref, wa_ref, wo_ref, wi_ref, brow_ref, deg_ref,
               a_ref, pg_ref):
    xv = x_ref[...]
    do = jnp.maximum(deg_ref[0], 1.0)
    di = jnp.maximum(deg_ref[1], 1.0)
    a_ref[...] = (jnp.dot(xv, wa_ref[...], preferred_element_type=jnp.float32)
                  + brow_ref[...])
    pg_ref[0, :, :] = jnp.dot(xv, wo_ref[...],
                              preferred_element_type=jnp.float32) / do
    pg_ref[1, :, :] = jnp.dot(xv, wi_ref[...],
                              preferred_element_type=jnp.float32) / di


_BLKP = 1280


def _proj_call(x, wa, wo, wi, brow, deg):
    return pl.pallas_call(
        _proj_body,
        grid=(NPAD // _BLKP,),
        in_specs=[
            pl.BlockSpec((_BLKP, DIN), lambda i: (i, 0)),
            pl.BlockSpec((DIN, 2 * DEMB), lambda i: (0, 0)),
            pl.BlockSpec((DIN, 2 * DEMB), lambda i: (0, 0)),
            pl.BlockSpec((DIN, 2 * DEMB), lambda i: (0, 0)),
            pl.BlockSpec((1, 2 * DEMB), lambda i: (0, 0)),
            pl.BlockSpec((2, _BLKP, 1), lambda i: (0, i, 0)),
        ],
        out_specs=[
            pl.BlockSpec((_BLKP, 2 * DEMB), lambda i: (i, 0)),
            pl.BlockSpec((2, _BLKP, 2 * DEMB), lambda i: (0, i, 0)),
        ],
        out_shape=[
            jax.ShapeDtypeStruct((NPAD, 2 * DEMB), jnp.float32),
            jax.ShapeDtypeStruct((2, NPAD, 2 * DEMB), jnp.float32),
        ],
    )(x, wa, wo, wi, brow, deg)


_M = 2
_D = 1
SCHUNK = 64
SROWS = EPAD // SCHUNK // NSUB
_NSTEP = SROWS


def _scatter_body(ei_hbm, pg_hbm, s_hbm,
                  idxs_v, idxd_v, rows_v, gsem, ssem, pgsrc_sh, acc_sh):
    c = lax.axis_index("c")
    s = lax.axis_index("s")
    row0 = s * SROWS
    pltpu.sync_copy(pg_hbm.at[pl.ds(c * NPAD + s * RPT, RPT)],
                    pgsrc_sh.at[pl.ds(s * RPT, RPT)])
    for m in range(_M):
        for r in range(SCHUNK):
            for g in range(2 * DEMB // LANES):
                rows_v[m, r, pl.ds(g * LANES, LANES)] = jnp.zeros(
                    (LANES,), jnp.float32)
    for q in range(RPT // SCHUNK):
        pltpu.sync_copy(rows_v.at[0],
                        acc_sh.at[pl.ds(s * RPT + q * SCHUNK, SCHUNK)])
    pltpu.sync_copy(ei_hbm.at[c, pl.ds(row0, SROWS)], idxs_v)
    pltpu.sync_copy(ei_hbm.at[1 - c, pl.ds(row0, SROWS)], idxd_v)
    plsc.subcore_barrier()

    def start_gather(k, m):
        pltpu.async_copy(pgsrc_sh.at[idxs_v.at[k]], rows_v.at[m], gsem.at[m])

    def wait_gather(m):
        pltpu.make_async_copy(pgsrc_sh.at[idxs_v.at[0]], rows_v.at[m],
                              gsem.at[m]).wait()

    def start_scatter(j, m):
        pltpu.async_copy(rows_v.at[m], acc_sh.at[idxd_v.at[j]], ssem.at[m],
                         add=True)

    def wait_scatter(m):
        pltpu.make_async_copy(rows_v.at[m], acc_sh.at[idxd_v.at[0]],
                              ssem.at[m]).wait()

    for b in range(_D):
        start_gather(b, b)
    for j in range(_D):
        start_gather(j + _D, j + _D)
        wait_gather(j)
        start_scatter(j, j)

    def steady(jj, carry):
        for b in range(_M):
            j = _D + jj * _M + b
            k = j + _D
            wait_scatter(b)
            start_gather(k, b)
            m = (_D + b) % _M
            wait_gather(m)
            start_scatter(j, m)
        return carry

    lax.fori_loop(0, (_NSTEP - 2 * _D) // _M, steady, 0)

    for j in range(_NSTEP - _D, _NSTEP):
        m = j % _M
        wait_gather(m)
        start_scatter(j, m)
    for m in range(_M):
        wait_scatter(m)

    plsc.subcore_barrier()
    for q in range(RPT // SCHUNK):
        sl = pl.ds(s * RPT + q * SCHUNK, SCHUNK)
        pltpu.sync_copy(acc_sh.at[sl], rows_v.at[0])
        pltpu.sync_copy(rows_v.at[0], s_hbm.at[c, sl])


_scatter_kernel = pl.kernel(
    _scatter_body,
    out_type=jax.ShapeDtypeStruct((NCORES, NPAD, 2 * DEMB), jnp.float32),
    mesh=plsc.VectorSubcoreMesh(**_SC_MESH),
    compiler_params=pltpu.CompilerParams(use_tc_tiling_on_sc=False,
                                         needs_layout_passes=False),
    scratch_types=[
        pltpu.VMEM((SROWS, SCHUNK), jnp.int32),
        pltpu.VMEM((SROWS, SCHUNK), jnp.int32),
        pltpu.VMEM((_M, SCHUNK, 2 * DEMB), jnp.float32),
        pltpu.SemaphoreType.DMA((_M,)),
        pltpu.SemaphoreType.DMA((_M,)),
        pltpu.VMEM_SHARED((NPAD, 2 * DEMB), jnp.float32),
        pltpu.VMEM_SHARED((NPAD, 2 * DEMB), jnp.float32),
    ],
)


def _out_body(a_ref, s_ref, wp_ref, bp_ref, out_ref, h_ref):
    g = a_ref[...] + s_ref[0] + s_ref[1]
    z = jax.nn.sigmoid(g[:, :DEMB])
    ht = jnp.tanh(g[:, DEMB:])
    h = jnp.maximum((1.0 - z) * ht, 0.0)
    h_ref[...] = h
    out_ref[...] = (jnp.dot(h, wp_ref[...], preferred_element_type=jnp.float32)
                    + bp_ref[...])


_BLKO = 2000


def _out_call(a, svec, wp, bprow):
    return pl.pallas_call(
        _out_body,
        grid=(NN // _BLKO,),
        in_specs=[
            pl.BlockSpec((_BLKO, 2 * DEMB), lambda i: (i, 0)),
            pl.BlockSpec((2, _BLKO, 2 * DEMB), lambda i: (0, i, 0)),
            pl.BlockSpec((DEMB, DTGT), lambda i: (0, 0)),
            pl.BlockSpec((1, DTGT), lambda i: (0, 0)),
        ],
        out_specs=[
            pl.BlockSpec((_BLKO, DTGT), lambda i: (i, 0)),
            pl.BlockSpec((_BLKO, DEMB), lambda i: (i, 0)),
        ],
        out_shape=[
            jax.ShapeDtypeStruct((NN, DTGT), jnp.float32),
            jax.ShapeDtypeStruct((NN, DEMB), jnp.float32),
        ],
    )(a, svec, wp, bprow)


def kernel(x, edge_index, Wz, bz, Wr, br, Wh, bh, Wp, bp):
    del Wr, br
    f32 = jnp.float32
    wa = jnp.concatenate([Wz[0, 0, :DIN] + Wz[1, 0, :DIN],
                          Wh[0, 0, :DIN] + Wh[1, 0, :DIN]], axis=1)
    wo = jnp.concatenate([Wz[0, 1, :DIN], Wh[0, 1, :DIN]], axis=1)
    wi = jnp.concatenate([Wz[1, 1, :DIN], Wh[1, 1, :DIN]], axis=1)
    brow = jnp.concatenate([bz, bh])[None, :]

    x_pad = jnp.pad(x, ((0, NPAD - NN), (0, 0)))
    ei_pad = jnp.pad(edge_index, ((0, 0), (0, EPAD - EE)),
                     constant_values=NN)
    ei3 = ei_pad.reshape(2, EROWS, CHUNK)
    deg = _deg_kernel(ei3).reshape(2, NPAD, 1)
    a, pg = _proj_call(x_pad, wa, wo, wi, brow, deg)
    ei3s = ei_pad.reshape(2, EPAD // SCHUNK, SCHUNK)
    svec = _scatter_kernel(ei3s, pg.reshape(2 * NPAD, 2 * DEMB))
    out, h = _out_call(a, svec, Wp, bp[None, :])
    return (out, h)


# speedup vs baseline: 1.5807x; 1.0002x over previous
import jax
import jax.numpy as jnp
from jax import lax
from jax.experimental import pallas as pl
from jax.experimental.pallas import tpu as pltpu
from jax.experimental.pallas import tpu_sc as plsc

NN = 10000
EE = 320000
DIN = 128
DEMB = 32
DTGT = 10
NCORES = 2
NSUB = 16
LANES = 16

NPAD = 10240
RPT = NPAD // NSUB
CHUNK = 128
ROWS_PER_TILE = 160
EPAD = NSUB * ROWS_PER_TILE * CHUNK
EROWS = EPAD // CHUNK

_SC_MESH = dict(core_axis_name="c", subcore_axis_name="s",
                num_cores=NCORES, num_subcores=NSUB)


DROWS = NPAD // LANES


def _deg_body(ei_hbm, deg_hbm, idx_v, acc_v, iota_v, deg_sh):
    c = lax.axis_index("c")
    s = lax.axis_index("s")
    pltpu.sync_copy(ei_hbm.at[c, pl.ds(s * ROWS_PER_TILE, ROWS_PER_TILE)], idx_v)

    def zrow(k, carry):
        acc_v[k, :] = jnp.zeros((LANES,), jnp.float32)
        return carry

    lax.fori_loop(0, DROWS, zrow, 0)
    zsl = pl.ds(s * (DROWS // NSUB), DROWS // NSUB)
    base = lax.iota(jnp.int32, LANES)
    for r in range(DROWS // CHUNK):
        for g in range(CHUNK // LANES):
            iota_v[r, pl.ds(g * LANES, LANES)] = base + (r * CHUNK + g * LANES)
    pltpu.sync_copy(acc_v.at[zsl], deg_sh.at[zsl])
    plsc.subcore_barrier()

    ones16 = jnp.ones((LANES,), jnp.float32)

    def hrow(r, carry):
        for g in range(CHUNK // LANES):
            n = idx_v[r, pl.ds(g * LANES, LANES)]
            plsc.addupdate_scatter(
                acc_v, [lax.shift_right_logical(n, 4),
                        lax.bitwise_and(n, 15)], ones16)
        return carry

    lax.fori_loop(0, ROWS_PER_TILE, hrow, 0)
    plsc.subcore_barrier()
    for r in range(DROWS // CHUNK):
        pltpu.sync_copy(acc_v.at[pl.ds(r * CHUNK, CHUNK)],
                        deg_sh.at[iota_v.at[r]], add=True)
    plsc.subcore_barrier()
    pltpu.sync_copy(deg_sh.at[zsl], acc_v.at[zsl])
    pltpu.sync_copy(acc_v.at[zsl], deg_hbm.at[c, zsl])


_deg_kernel = pl.kernel(
    _deg_body,
    out_type=jax.ShapeDtypeStruct((NCORES, DROWS, LANES), jnp.float32),
    mesh=plsc.VectorSubcoreMesh(**_SC_MESH),
    compiler_params=pltpu.CompilerParams(use_tc_tiling_on_sc=False,
                                         needs_layout_passes=False),
    scratch_types=[
        pltpu.VMEM((ROWS_PER_TILE, CHUNK), jnp.int32),
        pltpu.VMEM((DROWS, LANES), jnp.float32),
        pltpu.VMEM((DROWS // CHUNK, CHUNK), jnp.int32),
        pltpu.VMEM_SHARED((DROWS, LANES), jnp.float32),
    ],
)


def _proj_body(x_ref, wa_ref, wo_ref, wi_ref, brow_ref, deg_ref,
               a_ref, pg_ref):
    xv = x_ref[...]
    do = jnp.maximum(deg_ref[0], 1.0)
    di = jnp.maximum(deg_ref[1], 1.0)
    a_ref[...] = (jnp.dot(xv, wa_ref[...], preferred_element_type=jnp.float32)
                  + brow_ref[...])
    pg_ref[0, :, :] = jnp.dot(xv, wo_ref[...],
                              preferred_element_type=jnp.float32) / do
    pg_ref[1, :, :] = jnp.dot(xv, wi_ref[...],
                              preferred_element_type=jnp.float32) / di


_BLKP = 1280


def _proj_call(x, wa, wo, wi, brow, deg):
    return pl.pallas_call(
        _proj_body,
        grid=(NPAD // _BLKP,),
        in_specs=[
            pl.BlockSpec((_BLKP, DIN), lambda i: (i, 0)),
            pl.BlockSpec((DIN, 2 * DEMB), lambda i: (0, 0)),
            pl.BlockSpec((DIN, 2 * DEMB), lambda i: (0, 0)),
            pl.BlockSpec((DIN, 2 * DEMB), lambda i: (0, 0)),
            pl.BlockSpec((1, 2 * DEMB), lambda i: (0, 0)),
            pl.BlockSpec((2, _BLKP, 1), lambda i: (0, i, 0)),
        ],
        out_specs=[
            pl.BlockSpec((_BLKP, 2 * DEMB), lambda i: (i, 0)),
            pl.BlockSpec((2, _BLKP, 2 * DEMB), lambda i: (0, i, 0)),
        ],
        out_shape=[
            jax.ShapeDtypeStruct((NPAD, 2 * DEMB), jnp.float32),
            jax.ShapeDtypeStruct((2, NPAD, 2 * DEMB), jnp.float32),
        ],
    )(x, wa, wo, wi, brow, deg)


_M = 2
_D = 1
SCHUNK = 64
SROWS = EPAD // SCHUNK // NSUB
_NSTEP = SROWS


def _scatter_body(ei_hbm, pg_hbm, s_hbm,
                  idxs_v, idxd_v, rows_v, gsem, ssem, pgsrc_sh, acc_sh):
    c = lax.axis_index("c")
    s = lax.axis_index("s")
    row0 = s * SROWS
    pltpu.sync_copy(pg_hbm.at[pl.ds(c * NPAD + s * RPT, RPT)],
                    pgsrc_sh.at[pl.ds(s * RPT, RPT)])
    for m in range(_M):
        for r in range(SCHUNK):
            for g in range(2 * DEMB // LANES):
                rows_v[m, r, pl.ds(g * LANES, LANES)] = jnp.zeros(
                    (LANES,), jnp.float32)
    for q in range(RPT // SCHUNK):
        pltpu.sync_copy(rows_v.at[0],
                        acc_sh.at[pl.ds(s * RPT + q * SCHUNK, SCHUNK)])
    pltpu.sync_copy(ei_hbm.at[c, pl.ds(row0, SROWS)], idxs_v)
    pltpu.sync_copy(ei_hbm.at[1 - c, pl.ds(row0, SROWS)], idxd_v)
    plsc.subcore_barrier()

    def start_gather(k, m):
        pltpu.async_copy(pgsrc_sh.at[idxs_v.at[k]], rows_v.at[m], gsem.at[m])

    def wait_gather(m):
        pltpu.make_async_copy(pgsrc_sh.at[idxs_v.at[0]], rows_v.at[m],
                              gsem.at[m]).wait()

    def start_scatter(j, m):
        pltpu.async_copy(rows_v.at[m], acc_sh.at[idxd_v.at[j]], ssem.at[m],
                         add=True)

    def wait_scatter(m):
        pltpu.make_async_copy(rows_v.at[m], acc_sh.at[idxd_v.at[0]],
                              ssem.at[m]).wait()

    for b in range(_D):
        start_gather(b, b)
    for j in range(_D):
        start_gather(j + _D, j + _D)
        wait_gather(j)
        start_scatter(j, j)

    def steady(jj, carry):
        for b in range(_M):
            j = _D + jj * _M + b
            k = j + _D
            wait_scatter(b)
            start_gather(k, b)
            m = (_D + b) % _M
            wait_gather(m)
            start_scatter(j, m)
        return carry

    lax.fori_loop(0, (_NSTEP - 2 * _D) // _M, steady, 0)

    for j in range(_NSTEP - _D, _NSTEP):
        m = j % _M
        wait_gather(m)
        start_scatter(j, m)
    for m in range(_M):
        wait_scatter(m)

    plsc.subcore_barrier()
    for q in range(RPT // SCHUNK):
        sl = pl.ds(s * RPT + q * SCHUNK, SCHUNK)
        pltpu.sync_copy(acc_sh.at[sl], rows_v.at[0])
        pltpu.sync_copy(rows_v.at[0], s_hbm.at[c, sl])


_scatter_kernel = pl.kernel(
    _scatter_body,
    out_type=jax.ShapeDtypeStruct((NCORES, NPAD, 2 * DEMB), jnp.float32),
    mesh=plsc.VectorSubcoreMesh(**_SC_MESH),
    compiler_params=pltpu.CompilerParams(use_tc_tiling_on_sc=False,
                                         needs_layout_passes=False),
    scratch_types=[
        pltpu.VMEM((SROWS, SCHUNK), jnp.int32),
        pltpu.VMEM((SROWS, SCHUNK), jnp.int32),
        pltpu.VMEM((_M, SCHUNK, 2 * DEMB), jnp.float32),
        pltpu.SemaphoreType.DMA((_M,)),
        pltpu.SemaphoreType.DMA((_M,)),
        pltpu.VMEM_SHARED((NPAD, 2 * DEMB), jnp.float32),
        pltpu.VMEM_SHARED((NPAD, 2 * DEMB), jnp.float32),
    ],
)


def _out_body(a_ref, s_ref, wp_ref, bp_ref, out_ref, h_ref):
    g = a_ref[...] + s_ref[0] + s_ref[1]
    z = jax.nn.sigmoid(g[:, :DEMB])
    ht = jnp.tanh(g[:, DEMB:])
    h = jnp.maximum((1.0 - z) * ht, 0.0)
    h_ref[...] = h
    out_ref[...] = (jnp.dot(h, wp_ref[...], preferred_element_type=jnp.float32)
                    + bp_ref[...])


_BLKO = 2000


def _out_call(a, svec, wp, bprow):
    return pl.pallas_call(
        _out_body,
        grid=(NN // _BLKO,),
        in_specs=[
            pl.BlockSpec((_BLKO, 2 * DEMB), lambda i: (i, 0)),
            pl.BlockSpec((2, _BLKO, 2 * DEMB), lambda i: (0, i, 0)),
            pl.BlockSpec((DEMB, DTGT), lambda i: (0, 0)),
            pl.BlockSpec((1, DTGT), lambda i: (0, 0)),
        ],
        out_specs=[
            pl.BlockSpec((_BLKO, DTGT), lambda i: (i, 0)),
            pl.BlockSpec((_BLKO, DEMB), lambda i: (i, 0)),
        ],
        out_shape=[
            jax.ShapeDtypeStruct((NN, DTGT), jnp.float32),
            jax.ShapeDtypeStruct((NN, DEMB), jnp.float32),
        ],
    )(a, svec, wp, bprow)


def kernel(x, edge_index, Wz, bz, Wr, br, Wh, bh, Wp, bp):
    del Wr, br
    f32 = jnp.float32
    wa = jnp.concatenate([Wz[0, 0, :DIN] + Wz[1, 0, :DIN],
                          Wh[0, 0, :DIN] + Wh[1, 0, :DIN]], axis=1)
    wo = jnp.concatenate([Wz[0, 1, :DIN], Wh[0, 1, :DIN]], axis=1)
    wi = jnp.concatenate([Wz[1, 1, :DIN], Wh[1, 1, :DIN]], axis=1)
    brow = jnp.concatenate([bz, bh])[None, :]

    x_pad = jnp.pad(x, ((0, NPAD - NN), (0, 0)))
    ei_pad = jnp.pad(edge_index, ((0, 0), (0, EPAD - EE)),
                     constant_values=NN)
    ei3 = ei_pad.reshape(2, EROWS, CHUNK)
    deg = _deg_kernel(ei3).reshape(2, NPAD, 1)
    a, pg = _proj_call(x_pad, wa, wo, wi, brow, deg)
    ei3s = ei_pad.reshape(2, EPAD // SCHUNK, SCHUNK)
    svec = _scatter_kernel(ei3s, pg.reshape(2 * NPAD, 2 * DEMB))
    out, h = _out_call(a, svec, Wp, bp[None, :])
    return (out, h)
